# Initial kernel scaffold; baseline (speedup 1.0000x reference)
import jax
import jax.numpy as jnp
from jax.experimental import pallas as pl


def kernel(x, edge_index, batch, W1l, b1, W1r, W2l, b2, W2r, Wlin, blin):
    raise NotImplementedError("write your pallas kernel here")


# speedup vs baseline: 5.5335x; 5.5335x over previous
import functools

import jax
import jax.numpy as jnp
from jax import lax
from jax.experimental import pallas as pl
from jax.experimental.pallas import tpu as pltpu
from jax.experimental.pallas import tpu_sc as plsc

N = 10000
E = 320000
D = 128
G = 64
C = 16

NC = 2
NS = 16
NW = NC * NS
EPW = E // NW
K = 80
NCHUNK = EPW // K
NPAD = 10240
RPT = NPAD // NS

R = 1000
NBLK = N // R

def _mesh():
  return plsc.VectorSubcoreMesh(
      core_axis_name="c", subcore_axis_name="s", num_cores=NC, num_subcores=NS)


def _sc_edge_pass_deg(feat, src, dst, z128, z16, ones16):
  @functools.partial(
      pl.kernel,
      out_type=(jax.ShapeDtypeStruct((NC * NPAD, D), jnp.float32),
                jax.ShapeDtypeStruct((NC * NPAD,), jnp.float32)),
      mesh=_mesh(),
      scratch_types=[
          pltpu.VMEM((K,), jnp.int32),
          pltpu.VMEM((K,), jnp.int32),
          pltpu.VMEM((K, D), jnp.float32),
          pltpu.VMEM((K,), jnp.float32),
          pltpu.VMEM_SHARED((NPAD, D), jnp.float32),
          pltpu.VMEM_SHARED((NPAD,), jnp.float32),
          pltpu.SemaphoreType.DMA,
      ],
  )
  def k(feat_h, src_h, dst_h, z128_h, z16_h, ones_h, acc_out, deg_out,
        idx_s, idx_d, rows, ones_v, acc_sh, deg_sh, sem):
    c = lax.axis_index("c")
    s = lax.axis_index("s")
    wid = c * NS + s
    r0 = s * RPT
    pltpu.sync_copy(z128_h.at[pl.ds(r0, RPT)], acc_sh.at[pl.ds(r0, RPT)])
    pltpu.sync_copy(z16_h.at[pl.ds(r0, RPT)], deg_sh.at[pl.ds(r0, RPT)])
    pltpu.sync_copy(ones_h, ones_v)
    plsc.subcore_barrier()

    def chunk(i, carry):
      eb = wid * EPW + i * K
      pltpu.sync_copy(src_h.at[pl.ds(eb, K)], idx_s)
      pltpu.sync_copy(dst_h.at[pl.ds(eb, K)], idx_d)
      pltpu.async_copy(feat_h.at[idx_s], rows, sem).wait()
      pltpu.sync_copy(rows, acc_sh.at[idx_d], add=True)
      pltpu.sync_copy(ones_v, deg_sh.at[idx_d], add=True)
      return carry

    lax.fori_loop(0, NCHUNK, chunk, 0)
    plsc.subcore_barrier()
    pltpu.sync_copy(acc_sh.at[pl.ds(r0, RPT)],
                    acc_out.at[pl.ds(c * NPAD + r0, RPT)])
    pltpu.sync_copy(deg_sh.at[pl.ds(r0, RPT)],
                    deg_out.at[pl.ds(c * NPAD + r0, RPT)])

  return k(feat, src, dst, z128, z16, ones16)


def _sc_edge_pass(feat, src, dst, z128):
  @functools.partial(
      pl.kernel,
      out_type=jax.ShapeDtypeStruct((NC * NPAD, D), jnp.float32),
      mesh=_mesh(),
      scratch_types=[
          pltpu.VMEM((K,), jnp.int32),
          pltpu.VMEM((K,), jnp.int32),
          pltpu.VMEM((K, D), jnp.float32),
          pltpu.VMEM_SHARED((NPAD, D), jnp.float32),
          pltpu.SemaphoreType.DMA,
      ],
  )
  def k(feat_h, src_h, dst_h, z128_h, acc_out, idx_s, idx_d, rows, acc_sh,
        sem):
    c = lax.axis_index("c")
    s = lax.axis_index("s")
    wid = c * NS + s
    r0 = s * RPT
    pltpu.sync_copy(z128_h.at[pl.ds(r0, RPT)], acc_sh.at[pl.ds(r0, RPT)])
    plsc.subcore_barrier()

    def chunk(i, carry):
      eb = wid * EPW + i * K
      pltpu.sync_copy(src_h.at[pl.ds(eb, K)], idx_s)
      pltpu.sync_copy(dst_h.at[pl.ds(eb, K)], idx_d)
      pltpu.async_copy(feat_h.at[idx_s], rows, sem).wait()
      pltpu.sync_copy(rows, acc_sh.at[idx_d], add=True)
      return carry

    lax.fori_loop(0, NCHUNK, chunk, 0)
    plsc.subcore_barrier()
    pltpu.sync_copy(acc_sh.at[pl.ds(r0, RPT)],
                    acc_out.at[pl.ds(c * NPAD + r0, RPT)])

  return k(feat, src, dst, z128)


def _tc1_body(acc0, acc1, deg0, deg1, xb, w1lt, w1rt, b1r, out):
  deg = deg0[0, 0, :] + deg1[0, 0, :]
  deginv = (1.0 / jnp.maximum(deg, 1.0))[:, None]
  agg = (acc0[...] + acc1[...]) * deginv
  h = jnp.dot(agg, w1lt[...], preferred_element_type=jnp.float32)
  h = h + jnp.dot(xb[...], w1rt[...], preferred_element_type=jnp.float32)
  out[...] = jnp.maximum(h + b1r[...], 0.0)


def _tc_layer1(acc0, acc1, deg0, deg1, x, w1lt, w1rt, b1r):
  row_spec = pl.BlockSpec((R, D), lambda i: (i, 0))
  deg_spec = pl.BlockSpec((1, 1, R), lambda i: (i, 0, 0))
  w_spec = pl.BlockSpec((D, D), lambda i: (0, 0))
  b_spec = pl.BlockSpec((1, D), lambda i: (0, 0))
  return pl.pallas_call(
      _tc1_body,
      grid=(NBLK,),
      in_specs=[row_spec, row_spec, deg_spec, deg_spec, row_spec, w_spec,
                w_spec, b_spec],
      out_specs=row_spec,
      out_shape=jax.ShapeDtypeStruct((N, D), jnp.float32),
  )(acc0, acc1, deg0, deg1, x, w1lt, w1rt, b1r)


def _tc2_body(acc0, acc1, deg0, deg1, h1b, batchb, w2lt, w2rt, b2r, wlint,
              blinr, out, s_agg, s_h1, s_cnt):
  i = pl.program_id(0)

  @pl.when(i == 0)
  def _():
    s_agg[...] = jnp.zeros_like(s_agg)
    s_h1[...] = jnp.zeros_like(s_h1)
    s_cnt[...] = jnp.zeros_like(s_cnt)

  deg = deg0[0, 0, :] + deg1[0, 0, :]
  deginv = (1.0 / jnp.maximum(deg, 1.0))[:, None]
  agg = (acc0[...] + acc1[...]) * deginv
  ids = batchb[0, 0, :]
  onehot = (ids[:, None] == lax.broadcasted_iota(jnp.int32, (R, G), 1)
            ).astype(jnp.float32)
  dn = (((0,), (0,)), ((), ()))
  s_agg[...] += lax.dot_general(onehot, agg, dn,
                                preferred_element_type=jnp.float32)
  s_h1[...] += lax.dot_general(onehot, h1b[...], dn,
                               preferred_element_type=jnp.float32)
  s_cnt[...] += lax.dot_general(onehot, jnp.ones((R, D), jnp.float32), dn,
                                preferred_element_type=jnp.float32)

  @pl.when(i == pl.num_programs(0) - 1)
  def _():
    cnt = s_cnt[...]
    pooled = jnp.dot(s_agg[...], w2lt[...], preferred_element_type=jnp.float32)
    pooled += jnp.dot(s_h1[...], w2rt[...], preferred_element_type=jnp.float32)
    pooled = (pooled + cnt * b2r[...]) / jnp.maximum(cnt, 1.0)
    out[...] = jnp.dot(pooled, wlint[...],
                       preferred_element_type=jnp.float32) + blinr[...]


def _tc_layer2(acc0, acc1, deg0, deg1, h1, batch3, w2lt, w2rt, b2r, wlint,
               blinr):
  row_spec = pl.BlockSpec((R, D), lambda i: (i, 0))
  deg_spec = pl.BlockSpec((1, 1, R), lambda i: (i, 0, 0))
  batch_spec = pl.BlockSpec((1, 1, R), lambda i: (i, 0, 0))
  w_spec = pl.BlockSpec((D, D), lambda i: (0, 0))
  b_spec = pl.BlockSpec((1, D), lambda i: (0, 0))
  wlin_spec = pl.BlockSpec((D, C), lambda i: (0, 0))
  blin_spec = pl.BlockSpec((1, C), lambda i: (0, 0))
  out_spec = pl.BlockSpec((G, C), lambda i: (0, 0))
  return pl.pallas_call(
      _tc2_body,
      grid=(NBLK,),
      in_specs=[row_spec, row_spec, deg_spec, deg_spec, row_spec, batch_spec,
                w_spec, w_spec, b_spec, wlin_spec, blin_spec],
      out_specs=out_spec,
      out_shape=jax.ShapeDtypeStruct((G, C), jnp.float32),
      scratch_shapes=[
          pltpu.VMEM((G, D), jnp.float32),
          pltpu.VMEM((G, D), jnp.float32),
          pltpu.VMEM((G, D), jnp.float32),
      ],
  )(acc0, acc1, deg0, deg1, h1, batch3, w2lt, w2rt, b2r, wlint, blinr)


def kernel(x, edge_index, batch, W1l, b1, W1r, W2l, b2, W2r, Wlin, blin):
  src = edge_index[0]
  dst = edge_index[1]
  z128 = jnp.zeros((NPAD, D), jnp.float32)
  z1 = jnp.zeros((NPAD,), jnp.float32)
  ones1 = jnp.ones((K,), jnp.float32)

  acc1, deg = _sc_edge_pass_deg(x, src, dst, z128, z1, ones1)
  deg0 = deg[:N].reshape(NBLK, 1, R)
  deg1 = deg[NPAD:NPAD + N].reshape(NBLK, 1, R)
  h1 = _tc_layer1(acc1[:N], acc1[NPAD:NPAD + N], deg0, deg1, x,
                  W1l.T, W1r.T, b1.reshape(1, D))
  acc2 = _sc_edge_pass(h1, src, dst, z128)
  out = _tc_layer2(acc2[:N], acc2[NPAD:NPAD + N], deg0, deg1, h1,
                   batch.reshape(NBLK, 1, R), W2l.T, W2r.T,
                   b2.reshape(1, D), Wlin.T, blin.reshape(1, C))
  return out


# speedup vs baseline: 14.6856x; 2.6539x over previous
import functools

import jax
import jax.numpy as jnp
from jax import lax
from jax.experimental import pallas as pl
from jax.experimental.pallas import tpu as pltpu
from jax.experimental.pallas import tpu_sc as plsc

N = 10000
E = 320000
D = 128
G = 64
C = 16

NC = 2
NS = 16
NW = NC * NS
EPW = E // NW
K = 40
NCHUNK = EPW // K
NBUF = 5
NPAD = 10240
RPT = NPAD // NS

R = 1000
NBLK = N // R


def _mesh():
  return plsc.VectorSubcoreMesh(
      core_axis_name="c", subcore_axis_name="s", num_cores=NC, num_subcores=NS)


def _sc_edge_pass(feat, src, dst, z128, z1, ones1, with_deg):
  out_type = [jax.ShapeDtypeStruct((NC * NPAD, D), jnp.float32)]
  scratch = [
      pltpu.VMEM((EPW,), jnp.int32),
      [pltpu.VMEM((K,), jnp.int32) for _ in range(NBUF)],
      [pltpu.VMEM((K, D), jnp.float32) for _ in range(NBUF)],
      pltpu.VMEM((K,), jnp.float32),
      pltpu.VMEM_SHARED((NPAD, D), jnp.float32),
      pltpu.VMEM_SHARED((NPAD,), jnp.float32),
      pltpu.SemaphoreType.DMA((NBUF,)),
      pltpu.SemaphoreType.DMA((NBUF,)),
      pltpu.SemaphoreType.DMA((NBUF,)),
  ]
  if with_deg:
    out_type.append(jax.ShapeDtypeStruct((NC * NPAD,), jnp.float32))

  @functools.partial(
      pl.kernel,
      out_type=tuple(out_type) if with_deg else out_type[0],
      mesh=_mesh(),
      scratch_types=scratch,
  )
  def k(feat_h, src_h, dst_h, zf_h, z1_h, ones_h, *rest):
    if with_deg:
      (acc_out, deg_out, src_all, dstb, rows, ones_v, acc_sh, deg_sh,
       gsem, ssem, dsem) = rest
    else:
      (acc_out, src_all, dstb, rows, ones_v, acc_sh, deg_sh,
       gsem, ssem, dsem) = rest
      deg_out = None
    cc = lax.axis_index("c")
    ss = lax.axis_index("s")
    wid = cc * NS + ss
    r0 = ss * RPT
    e0 = wid * EPW
    pltpu.sync_copy(zf_h.at[pl.ds(r0, RPT)], acc_sh.at[pl.ds(r0, RPT)])
    if with_deg:
      pltpu.sync_copy(z1_h.at[pl.ds(r0, RPT)], deg_sh.at[pl.ds(r0, RPT)])
      pltpu.sync_copy(ones_h, ones_v)
    pltpu.sync_copy(src_h.at[pl.ds(e0, EPW)], src_all)
    plsc.subcore_barrier()

    def fire(i, b):
      pltpu.async_copy(dst_h.at[pl.ds(e0 + i * K, K)], dstb[b], gsem.at[b])
      pltpu.async_copy(feat_h.at[src_all.at[pl.ds(i * K, K)]], rows[b],
                       gsem.at[b])

    for b in range(NBUF):
      fire(b, b)

    @pl.loop(0, NCHUNK, step=NBUF)
    def _(t):
      for b in range(NBUF):
        i = t + b
        pltpu.make_async_copy(dst_h.at[pl.ds(e0 + i * K, K)], dstb[b],
                              gsem.at[b]).wait()
        pltpu.make_async_copy(feat_h.at[src_all.at[pl.ds(i * K, K)]],
                              rows[b], gsem.at[b]).wait()
        pltpu.async_copy(rows[b], acc_sh.at[dstb[b]], ssem.at[b], add=True)
        if with_deg:
          pltpu.async_copy(ones_v, deg_sh.at[dstb[b]], dsem.at[b], add=True)

        @pl.when(i + NBUF < NCHUNK)
        def _():
          pltpu.make_async_copy(rows[b], acc_sh.at[dstb[b]],
                                ssem.at[b]).wait()
          if with_deg:
            pltpu.make_async_copy(ones_v, deg_sh.at[dstb[b]],
                                  dsem.at[b]).wait()
          fire(i + NBUF, b)

    for b in range(NBUF):
      pltpu.make_async_copy(rows[b], acc_sh.at[dstb[b]], ssem.at[b]).wait()
      if with_deg:
        pltpu.make_async_copy(ones_v, deg_sh.at[dstb[b]], dsem.at[b]).wait()
    plsc.subcore_barrier()
    pltpu.sync_copy(acc_sh.at[pl.ds(r0, RPT)],
                    acc_out.at[pl.ds(cc * NPAD + r0, RPT)])
    if with_deg:
      pltpu.sync_copy(deg_sh.at[pl.ds(r0, RPT)],
                      deg_out.at[pl.ds(cc * NPAD + r0, RPT)])

  return k(feat, src, dst, z128, z1, ones1)


def _tc1_body(acc0, acc1, deg0, deg1, xb, w1lt, w1rt, b1r, out):
  deg = deg0[0, 0, :] + deg1[0, 0, :]
  deginv = (1.0 / jnp.maximum(deg, 1.0))[:, None]
  agg = (acc0[...] + acc1[...]) * deginv
  h = jnp.dot(agg, w1lt[...], preferred_element_type=jnp.float32)
  h = h + jnp.dot(xb[...], w1rt[...], preferred_element_type=jnp.float32)
  out[...] = jnp.maximum(h + b1r[...], 0.0)


def _tc_layer1(acc0, acc1, deg0, deg1, x, w1lt, w1rt, b1r):
  row_spec = pl.BlockSpec((R, D), lambda i: (i, 0))
  deg_spec = pl.BlockSpec((1, 1, R), lambda i: (i, 0, 0))
  w_spec = pl.BlockSpec((D, D), lambda i: (0, 0))
  b_spec = pl.BlockSpec((1, D), lambda i: (0, 0))
  return pl.pallas_call(
      _tc1_body,
      grid=(NBLK,),
      in_specs=[row_spec, row_spec, deg_spec, deg_spec, row_spec, w_spec,
                w_spec, b_spec],
      out_specs=row_spec,
      out_shape=jax.ShapeDtypeStruct((N, D), jnp.float32),
  )(acc0, acc1, deg0, deg1, x, w1lt, w1rt, b1r)


def _tc2_body(acc0, acc1, deg0, deg1, h1b, batchb, w2lt, w2rt, b2r, wlint,
              blinr, out, s_agg, s_h1, s_cnt):
  i = pl.program_id(0)

  @pl.when(i == 0)
  def _():
    s_agg[...] = jnp.zeros_like(s_agg)
    s_h1[...] = jnp.zeros_like(s_h1)
    s_cnt[...] = jnp.zeros_like(s_cnt)

  deg = deg0[0, 0, :] + deg1[0, 0, :]
  deginv = (1.0 / jnp.maximum(deg, 1.0))[:, None]
  agg = (acc0[...] + acc1[...]) * deginv
  ids = batchb[0, 0, :]
  onehot = (ids[:, None] == lax.broadcasted_iota(jnp.int32, (R, G), 1)
            ).astype(jnp.float32)
  dn = (((0,), (0,)), ((), ()))
  s_agg[...] += lax.dot_general(onehot, agg, dn,
                                preferred_element_type=jnp.float32)
  s_h1[...] += lax.dot_general(onehot, h1b[...], dn,
                               preferred_element_type=jnp.float32)
  s_cnt[...] += lax.dot_general(onehot, jnp.ones((R, D), jnp.float32), dn,
                                preferred_element_type=jnp.float32)

  @pl.when(i == pl.num_programs(0) - 1)
  def _():
    cnt = s_cnt[...]
    pooled = jnp.dot(s_agg[...], w2lt[...], preferred_element_type=jnp.float32)
    pooled += jnp.dot(s_h1[...], w2rt[...], preferred_element_type=jnp.float32)
    pooled = (pooled + cnt * b2r[...]) / jnp.maximum(cnt, 1.0)
    out[...] = jnp.dot(pooled, wlint[...],
                       preferred_element_type=jnp.float32) + blinr[...]


def _tc_layer2(acc0, acc1, deg0, deg1, h1, batch3, w2lt, w2rt, b2r, wlint,
               blinr):
  row_spec = pl.BlockSpec((R, D), lambda i: (i, 0))
  deg_spec = pl.BlockSpec((1, 1, R), lambda i: (i, 0, 0))
  batch_spec = pl.BlockSpec((1, 1, R), lambda i: (i, 0, 0))
  w_spec = pl.BlockSpec((D, D), lambda i: (0, 0))
  b_spec = pl.BlockSpec((1, D), lambda i: (0, 0))
  wlin_spec = pl.BlockSpec((D, C), lambda i: (0, 0))
  blin_spec = pl.BlockSpec((1, C), lambda i: (0, 0))
  out_spec = pl.BlockSpec((G, C), lambda i: (0, 0))
  return pl.pallas_call(
      _tc2_body,
      grid=(NBLK,),
      in_specs=[row_spec, row_spec, deg_spec, deg_spec, row_spec, batch_spec,
                w_spec, w_spec, b_spec, wlin_spec, blin_spec],
      out_specs=out_spec,
      out_shape=jax.ShapeDtypeStruct((G, C), jnp.float32),
      scratch_shapes=[
          pltpu.VMEM((G, D), jnp.float32),
          pltpu.VMEM((G, D), jnp.float32),
          pltpu.VMEM((G, D), jnp.float32),
      ],
  )(acc0, acc1, deg0, deg1, h1, batch3, w2lt, w2rt, b2r, wlint, blinr)


def kernel(x, edge_index, batch, W1l, b1, W1r, W2l, b2, W2r, Wlin, blin):
  src = edge_index[0]
  dst = edge_index[1]
  z128 = jnp.zeros((NPAD, D), jnp.float32)
  z1 = jnp.zeros((NPAD,), jnp.float32)
  ones1 = jnp.ones((K,), jnp.float32)

  acc1, deg = _sc_edge_pass(x, src, dst, z128, z1, ones1, with_deg=True)
  deg0 = deg[:N].reshape(NBLK, 1, R)
  deg1 = deg[NPAD:NPAD + N].reshape(NBLK, 1, R)
  h1 = _tc_layer1(acc1[:N], acc1[NPAD:NPAD + N], deg0, deg1, x,
                  W1l.T, W1r.T, b1.reshape(1, D))
  acc2 = _sc_edge_pass(h1, src, dst, z128, z1, ones1, with_deg=False)
  out = _tc_layer2(acc2[:N], acc2[NPAD:NPAD + N], deg0, deg1, h1,
                   batch.reshape(NBLK, 1, R), W2l.T, W2r.T,
                   b2.reshape(1, D), Wlin.T, blin.reshape(1, C))
  return out
